# baseline (speedup 1.0000x reference)
import functools

import jax
import jax.numpy as jnp
from jax import lax
from jax.experimental import pallas as pl
from jax.experimental.pallas import tpu as pltpu
from jax.experimental.pallas import tpu_sc as plsc

NC = 2
NS = 16
NW = NC * NS
B = 4096
D = 64
PLEN = 20
PPAD = 32
BPW = B // NW


def _sc_gather_body(cw_hbm, p0idx_hbm, w0_hbm, w1_hbm, cvec_hbm, p0_hbm,
                    idx_v, rows_v, pidx_v, prow_v, sem, psem):
    wid = lax.axis_index("s") * NC + lax.axis_index("c")
    base = wid * BPW
    pltpu.sync_copy(cw_hbm.at[pl.ds(base, BPW)], idx_v)
    pltpu.async_copy(w0_hbm.at[idx_v], rows_v, sem).wait()
    pltpu.sync_copy(rows_v, cvec_hbm.at[pl.ds(base, BPW)])

    @pl.when(wid == 0)
    def _():
        pltpu.sync_copy(p0idx_hbm, pidx_v)
        pltpu.async_copy(w1_hbm.at[pidx_v], prow_v, psem).wait()
        pltpu.sync_copy(prow_v, p0_hbm)


def _sc_gather(c_words, paths0_pad, W0, W1):
    mesh = plsc.VectorSubcoreMesh(core_axis_name="c", subcore_axis_name="s",
                                  num_cores=NC, num_subcores=NS)
    out_type = (jax.ShapeDtypeStruct((B, D), jnp.float32),
                jax.ShapeDtypeStruct((PPAD, D), jnp.float32))
    scratch = [
        pltpu.VMEM((BPW,), jnp.int32),
        pltpu.VMEM((BPW, D), jnp.float32),
        pltpu.VMEM((PPAD,), jnp.int32),
        pltpu.VMEM((PPAD, D), jnp.float32),
        pltpu.SemaphoreType.DMA,
        pltpu.SemaphoreType.DMA,
    ]
    return pl.kernel(_sc_gather_body, out_type=out_type, mesh=mesh,
                     scratch_types=scratch,
                     compiler_params=pltpu.CompilerParams(
                         use_tc_tiling_on_sc=False))(
                             c_words, paths0_pad, W0, W1)


def _tc_loss_body(cvec_ref, p0_ref, labels_ref, out_ref):
    c = cvec_ref[...]
    p = p0_ref[...]
    scores = lax.dot_general(c, p, (((1,), (1,)), ((), ())),
                             preferred_element_type=jnp.float32)
    lab = labels_ref[...]
    z = jnp.log(1.0 / (1.0 + jnp.exp(-scores)))
    log_z = jnp.maximum(jnp.log(z), -100.0)
    log_1mz = jnp.maximum(jnp.log(1.0 - z), -100.0)
    term = lab * log_z + (1.0 - lab) * log_1mz
    col = lax.broadcasted_iota(jnp.int32, (B, PPAD), 1)
    term = jnp.where(col < PLEN, term, 0.0)
    out_ref[0, 0] = -jnp.sum(term)


def _tc_loss(c_vec, p0, labels_pad):
    out = pl.pallas_call(
        _tc_loss_body,
        out_shape=jax.ShapeDtypeStruct((1, 1), jnp.float32),
        out_specs=pl.BlockSpec(memory_space=pltpu.SMEM),
    )(c_vec, p0, labels_pad)
    return out[0, 0]


def kernel(c_words, paths, labels, W0, W1):
    c_words = jnp.squeeze(c_words).astype(jnp.int32)
    paths0 = jnp.squeeze(paths)[0].astype(jnp.int32)
    paths0_pad = jnp.pad(paths0, (0, PPAD - PLEN))
    labels_pad = jnp.pad(jnp.squeeze(labels), ((0, 0), (0, PPAD - PLEN)))
    c_vec, p0 = _sc_gather(c_words, paths0_pad, W0, W1)
    return _tc_loss(c_vec, p0, labels_pad)


# speedup vs baseline: 1.5942x; 1.5942x over previous
import jax
import jax.numpy as jnp
from jax import lax
from jax.experimental import pallas as pl
from jax.experimental.pallas import tpu as pltpu

B = 4096
D = 64
PLEN = 20
PPAD = 32
CHUNK = 512


def _body(cw_ref, p0i_ref, w0_ref, w1_ref, labels_ref, out_ref,
          rows, p0b, sem, psem):
    for j in range(PLEN):
        pltpu.make_async_copy(w1_ref.at[pl.ds(p0i_ref[j], 1)],
                              p0b.at[pl.ds(j, 1)], psem).start()

    def issue(b, _):
        pltpu.make_async_copy(w0_ref.at[pl.ds(cw_ref[b], 1)],
                              rows.at[pl.ds(b, 1)], sem).start()
        return 0

    lax.fori_loop(0, B, issue, 0, unroll=8)
    pltpu.make_async_copy(w0_ref.at[pl.ds(0, B)], rows, sem).wait()
    pltpu.make_async_copy(w1_ref.at[pl.ds(0, PLEN)],
                          p0b.at[pl.ds(0, PLEN)], psem).wait()

    c = rows[...]
    p = p0b[...]
    scores = lax.dot_general(c, p, (((1,), (1,)), ((), ())),
                             preferred_element_type=jnp.float32)
    s = scores[:, :PLEN]
    lab = labels_ref[...]
    z = jnp.log(1.0 / (1.0 + jnp.exp(-s)))
    log_z = jnp.maximum(jnp.log(z), -100.0)
    log_1mz = jnp.maximum(jnp.log(1.0 - z), -100.0)
    out_ref[0, 0] = -jnp.sum(lab * log_z + (1.0 - lab) * log_1mz)


def kernel(c_words, paths, labels, W0, W1):
    c_words = jnp.squeeze(c_words).astype(jnp.int32)
    paths0 = jnp.squeeze(paths)[0].astype(jnp.int32)
    labels = jnp.squeeze(labels)
    out = pl.pallas_call(
        _body,
        out_shape=jax.ShapeDtypeStruct((1, 1), jnp.float32),
        in_specs=[
            pl.BlockSpec(memory_space=pltpu.SMEM),
            pl.BlockSpec(memory_space=pltpu.SMEM),
            pl.BlockSpec(memory_space=pl.ANY),
            pl.BlockSpec(memory_space=pl.ANY),
            pl.BlockSpec(memory_space=pltpu.VMEM),
        ],
        out_specs=pl.BlockSpec(memory_space=pltpu.SMEM),
        scratch_shapes=[
            pltpu.VMEM((B, D), jnp.float32),
            pltpu.VMEM((PPAD, D), jnp.float32),
            pltpu.SemaphoreType.DMA,
            pltpu.SemaphoreType.DMA,
        ],
    )(c_words, paths0, W0, W1, labels)
    return out[0, 0]
